# baseline (speedup 1.0000x reference)
import functools

import jax
import jax.numpy as jnp
from jax import lax
from jax.experimental import pallas as pl
from jax.experimental.pallas import tpu as pltpu
from jax.experimental.pallas import tpu_sc as plsc

D_MODEL = 64
CHUNK = 128


def _make_gather(B, D, n_workers, nc):
  b_per_w = B // n_workers
  n_chunks = b_per_w // CHUNK
  mesh = plsc.VectorSubcoreMesh(core_axis_name="c", subcore_axis_name="s")

  @functools.partial(
      pl.kernel,
      mesh=mesh,
      out_type=jax.ShapeDtypeStruct((B, D), jnp.float32),
      scratch_types=[
          pltpu.VMEM((CHUNK,), jnp.int32),
          pltpu.VMEM((CHUNK, D), jnp.float32),
          pltpu.SemaphoreType.DMA,
      ],
      compiler_params=pltpu.CompilerParams(use_tc_tiling_on_sc=False),
  )
  def gather_kernel(idx_hbm, table_hbm, out_hbm, idx_v, rows_v, sem):
    wid = lax.axis_index("s") * nc + lax.axis_index("c")
    base = wid * b_per_w

    def step(j, carry):
      off = base + j * CHUNK
      pltpu.sync_copy(idx_hbm.at[pl.ds(off, CHUNK)], idx_v)
      pltpu.async_copy(table_hbm.at[idx_v], rows_v, sem).wait()
      pltpu.sync_copy(rows_v, out_hbm.at[pl.ds(off, CHUNK)])
      return carry

    lax.fori_loop(0, n_chunks, step, 0)

  return gather_kernel


@jax.jit
def kernel(x, table):
  B0, T = x.shape
  B = B0 * T
  info = plsc.get_sparse_core_info()
  nw = info.num_cores * info.num_subcores
  idx = jnp.asarray(x, jnp.int32).reshape(B)
  out = _make_gather(B, D_MODEL, nw, info.num_cores)(idx, table)
  return out.reshape(B0, T, D_MODEL)


# speedup vs baseline: 1.2018x; 1.2018x over previous
import functools

import jax
import jax.numpy as jnp
from jax import lax
from jax.experimental import pallas as pl
from jax.experimental.pallas import tpu as pltpu
from jax.experimental.pallas import tpu_sc as plsc

D_MODEL = 64
CHUNK = 128
SC_ROWS = 512
SUB = SC_ROWS // CHUNK
NBUF = 2


def _make_gather(B, D, n_workers, nc):
  b_per_w = B // n_workers
  n_sc = b_per_w // SC_ROWS
  mesh = plsc.VectorSubcoreMesh(core_axis_name="c", subcore_axis_name="s")

  @functools.partial(
      pl.kernel,
      mesh=mesh,
      out_type=jax.ShapeDtypeStruct((B, D), jnp.float32),
      scratch_types=[
          pltpu.VMEM((b_per_w,), jnp.int32),
          pltpu.VMEM((NBUF, SC_ROWS, D), jnp.float32),
          pltpu.SemaphoreType.DMA,
          pltpu.SemaphoreType.DMA,
          pltpu.SemaphoreType.DMA,
          pltpu.SemaphoreType.DMA,
      ],
      compiler_params=pltpu.CompilerParams(use_tc_tiling_on_sc=False),
  )
  def gather_kernel(idx_hbm, table_hbm, out_hbm, idx_v, rows_v, g0, g1, o0,
                    o1):
    wid = lax.axis_index("s") * nc + lax.axis_index("c")
    base = wid * b_per_w
    gsems = [g0, g1]
    osems = [o0, o1]

    pltpu.sync_copy(idx_hbm.at[pl.ds(base, b_per_w)], idx_v)

    def issue_gathers(j, half):
      for u in range(SUB):
        pltpu.async_copy(
            table_hbm.at[idx_v.at[pl.ds(j * SC_ROWS + u * CHUNK, CHUNK)]],
            rows_v.at[half, pl.ds(u * CHUNK, CHUNK)],
            gsems[half],
        )

    def drain_gathers(half):
      for u in range(SUB):
        pltpu.make_async_copy(
            table_hbm.at[idx_v.at[pl.ds(u * CHUNK, CHUNK)]],
            rows_v.at[half, pl.ds(u * CHUNK, CHUNK)],
            gsems[half],
        ).wait()

    def drain_write(half):
      pltpu.make_async_copy(
          out_hbm.at[pl.ds(base, SC_ROWS)],
          rows_v.at[half],
          osems[half],
      ).wait()

    issue_gathers(0, 0)
    issue_gathers(1, 1)

    def pair(p, carry):
      for half in range(NBUF):
        j = NBUF * p + half
        drain_gathers(half)
        pltpu.async_copy(
            rows_v.at[half],
            out_hbm.at[pl.ds(base + j * SC_ROWS, SC_ROWS)],
            osems[half],
        )

        @pl.when(j + NBUF < n_sc)
        def _():
          drain_write(half)
          issue_gathers(j + NBUF, half)

      return carry

    lax.fori_loop(0, n_sc // NBUF, pair, 0)
    drain_write(0)
    drain_write(1)

  return gather_kernel


@jax.jit
def kernel(x, table):
  B0, T = x.shape
  B = B0 * T
  info = plsc.get_sparse_core_info()
  nw = info.num_cores * info.num_subcores
  idx = jnp.asarray(x, jnp.int32).reshape(B)
  out = _make_gather(B, D_MODEL, nw, info.num_cores)(idx, table)
  return out.reshape(B0, T, D_MODEL)
